# Initial kernel scaffold; baseline (speedup 1.0000x reference)
import jax
import jax.numpy as jnp
from jax.experimental import pallas as pl


def kernel(input, edge_index, adj_values, W):
    raise NotImplementedError("write your pallas kernel here")


# speedup vs baseline: 3.9215x; 3.9215x over previous
import functools

import jax
import jax.numpy as jnp
from jax import lax
from jax.experimental import pallas as pl
from jax.experimental.pallas import tpu as pltpu
from jax.experimental.pallas import tpu_sc as plsc

NC = 2
NS = 16
NW = NC * NS
L = 16
C = 128


def _mm_body(x_ref, w_ref, o_ref):
    o_ref[...] = jnp.dot(x_ref[...], w_ref[...],
                         preferred_element_type=jnp.float32)


def _finish_body(p_ref, o_ref):
    n = o_ref.shape[0]
    s = p_ref[0, :n, :] + p_ref[1, :n, :]
    o_ref[...] = jnp.where(s >= 0.0, s, 0.01 * s)


def kernel(input, edge_index, adj_values, W):
    n, d_in = input.shape
    d_out = W.shape[1]
    e = edge_index.shape[1]
    fslices = d_out // L

    support = pl.pallas_call(
        _mm_body,
        out_shape=jax.ShapeDtypeStruct((n, d_out), jnp.float32),
    )(input, W)

    per_tile = pl.cdiv(e, NW * C) * C
    e_pad = per_tile * NW
    pad = e_pad - e
    src = jnp.concatenate([edge_index[1], jnp.zeros((pad,), jnp.int32)])
    dst = jnp.concatenate([edge_index[0], jnp.zeros((pad,), jnp.int32)])
    val = jnp.concatenate([adj_values, jnp.zeros((pad,), jnp.float32)])

    rows_per_tile = pl.cdiv(pl.cdiv(n, NS), C) * C
    n_acc = rows_per_tile * NS
    wb_chunks = rows_per_tile // C
    n_chunks = per_tile // C

    mesh = plsc.VectorSubcoreMesh(core_axis_name="c", subcore_axis_name="s")

    @functools.partial(
        pl.kernel,
        out_type=jax.ShapeDtypeStruct((NC, n_acc, d_out), jnp.float32),
        mesh=mesh,
        scratch_types=[
            pltpu.VMEM((C,), jnp.int32),
            pltpu.VMEM((C,), jnp.int32),
            pltpu.VMEM((C,), jnp.float32),
            pltpu.VMEM((C, d_out), jnp.float32),
            pltpu.VMEM_SHARED((n_acc, d_out), jnp.float32),
            pltpu.SemaphoreType.DMA,
        ],
    )
    def sc_scatter(sup_hbm, src_hbm, dst_hbm, val_hbm, out_hbm,
                   idx_v, dst_v, val_v, rows_v, acc_sh, sem):
        cid = lax.axis_index("c")
        sid = lax.axis_index("s")
        wid = cid * NS + sid
        row0 = sid * rows_per_tile

        z = jnp.zeros((L,), jnp.float32)

        def zero_body(i, carry):
            for f in range(fslices):
                rows_v[i, pl.ds(f * L, L)] = z
            return carry

        lax.fori_loop(0, C, zero_body, 0)
        for k in range(wb_chunks):
            pltpu.sync_copy(rows_v, acc_sh.at[pl.ds(row0 + k * C, C)])
        plsc.subcore_barrier()

        base_tile = wid * per_tile

        def chunk_body(j, carry):
            base = base_tile + j * C
            pltpu.sync_copy(src_hbm.at[pl.ds(base, C)], idx_v)
            pltpu.sync_copy(dst_hbm.at[pl.ds(base, C)], dst_v)
            pltpu.sync_copy(val_hbm.at[pl.ds(base, C)], val_v)
            pltpu.async_copy(sup_hbm.at[idx_v], rows_v, sem).wait()

            def scale_group(g, gc):
                vals = val_v[pl.ds(g * L, L)]
                for ei in range(L):
                    vb = jnp.full((L,), vals[ei], jnp.float32)
                    row = g * L + ei
                    for f in range(fslices):
                        sl = pl.ds(f * L, L)
                        rows_v[row, sl] = rows_v[row, sl] * vb
                return gc

            lax.fori_loop(0, C // L, scale_group, 0)
            pltpu.sync_copy(rows_v, acc_sh.at[dst_v], add=True)
            return carry

        lax.fori_loop(0, n_chunks, chunk_body, 0)
        plsc.subcore_barrier()

        for k in range(wb_chunks):
            r = row0 + k * C
            pltpu.sync_copy(acc_sh.at[pl.ds(r, C)], rows_v)
            pltpu.sync_copy(rows_v, out_hbm.at[cid, pl.ds(r, C)])

    partials = sc_scatter(support, src, dst, val)

    return pl.pallas_call(
        _finish_body,
        out_shape=jax.ShapeDtypeStruct((n, d_out), jnp.float32),
    )(partials)


# speedup vs baseline: 4.0995x; 1.0454x over previous
import functools

import jax
import jax.numpy as jnp
from jax import lax
from jax.experimental import pallas as pl
from jax.experimental.pallas import tpu as pltpu
from jax.experimental.pallas import tpu_sc as plsc

NC = 2
NS = 16
NW = NC * NS
L = 16
C = 128


def _mm_body(x_ref, w_ref, o_ref):
    o_ref[...] = jnp.dot(x_ref[...], w_ref[...],
                         preferred_element_type=jnp.float32)


def _finish_body(p_ref, o_ref):
    n = o_ref.shape[0]
    s = p_ref[0, :n, :] + p_ref[1, :n, :]
    o_ref[...] = jnp.where(s >= 0.0, s, 0.01 * s)


def kernel(input, edge_index, adj_values, W):
    n, d_in = input.shape
    d_out = W.shape[1]
    e = edge_index.shape[1]
    fslices = d_out // L

    support = pl.pallas_call(
        _mm_body,
        out_shape=jax.ShapeDtypeStruct((n, d_out), jnp.float32),
    )(input, W)

    n_chunks = pl.cdiv(pl.cdiv(e, NW * C), 4) * 4
    per_tile = n_chunks * C
    e_pad = per_tile * NW
    pad = e_pad - e
    src = jnp.concatenate([edge_index[1], jnp.zeros((pad,), jnp.int32)])
    dst = jnp.concatenate([edge_index[0], jnp.zeros((pad,), jnp.int32)])
    val = jnp.concatenate([adj_values, jnp.zeros((pad,), jnp.float32)])
    edges = jnp.stack(
        [src.reshape(NW, n_chunks, C),
         dst.reshape(NW, n_chunks, C)], axis=2)
    vals3 = val.reshape(NW, n_chunks, C)

    rows_per_tile = pl.cdiv(pl.cdiv(n, NS), C) * C
    n_acc = rows_per_tile * NS
    wb_chunks = rows_per_tile // C

    mesh = plsc.VectorSubcoreMesh(core_axis_name="c", subcore_axis_name="s")

    @functools.partial(
        pl.kernel,
        out_type=jax.ShapeDtypeStruct((NC, n_acc, d_out), jnp.float32),
        mesh=mesh,
        scratch_types=[
            pltpu.VMEM((2, C), jnp.int32),
            pltpu.VMEM((2, C), jnp.int32),
            pltpu.VMEM((2, C), jnp.int32),
            pltpu.VMEM((2, C), jnp.int32),
            pltpu.VMEM((C,), jnp.float32),
            pltpu.VMEM((C,), jnp.float32),
            pltpu.VMEM((C,), jnp.float32),
            pltpu.VMEM((C,), jnp.float32),
            pltpu.VMEM((C, d_out), jnp.float32),
            pltpu.VMEM((C, d_out), jnp.float32),
            pltpu.VMEM_SHARED((n_acc, d_out), jnp.float32),
            pltpu.SemaphoreType.DMA,
            pltpu.SemaphoreType.DMA,
            pltpu.SemaphoreType.DMA,
            pltpu.SemaphoreType.DMA,
            pltpu.SemaphoreType.DMA,
            pltpu.SemaphoreType.DMA,
            pltpu.SemaphoreType.DMA,
            pltpu.SemaphoreType.DMA,
        ],
    )
    def sc_scatter(sup_hbm, edges_hbm, vals_hbm, out_hbm,
                   eb0, eb1, eb2, eb3, vb0, vb1, vb2, vb3, rw0, rw1, acc_sh,
                   es0, es1, es2, es3, gs0, gs1, ss0, ss1):
        ebufs = [eb0, eb1, eb2, eb3]
        vbufs = [vb0, vb1, vb2, vb3]
        rows = [rw0, rw1]
        esem = [es0, es1, es2, es3]
        gsem = [gs0, gs1]
        ssem = [ss0, ss1]

        cid = lax.axis_index("c")
        sid = lax.axis_index("s")
        wid = cid * NS + sid
        row0 = sid * rows_per_tile
        last = n_chunks - 1

        z = jnp.zeros((L,), jnp.float32)

        def zero_body(i, carry):
            for f in range(fslices):
                rw0[i, pl.ds(f * L, L)] = z
            return carry

        lax.fori_loop(0, C, zero_body, 0)
        for k in range(wb_chunks):
            pltpu.sync_copy(rw0, acc_sh.at[pl.ds(row0 + k * C, C)])
        plsc.subcore_barrier()

        def edge_dma(chunk, slot):
            pltpu.async_copy(edges_hbm.at[wid, chunk], ebufs[slot],
                             esem[slot])
            pltpu.async_copy(vals_hbm.at[wid, chunk], vbufs[slot],
                             esem[slot])

        def gather_dma(slot, rslot):
            return pltpu.async_copy(sup_hbm.at[ebufs[slot].at[0]],
                                    rows[rslot], gsem[rslot])

        def scatter_dma(slot, rslot):
            return pltpu.async_copy(rows[rslot],
                                    acc_sh.at[ebufs[slot].at[1]],
                                    ssem[rslot], add=True)

        def edge_wait(slot):
            pltpu.make_async_copy(edges_hbm.at[wid, 0], ebufs[slot],
                                  esem[slot]).wait()
            pltpu.make_async_copy(vals_hbm.at[wid, 0], vbufs[slot],
                                  esem[slot]).wait()

        def gather_wait(rslot):
            pltpu.make_async_copy(sup_hbm.at[ebufs[0].at[0]], rows[rslot],
                                  gsem[rslot]).wait()

        def scatter_wait(rslot):
            pltpu.make_async_copy(rows[rslot], acc_sh.at[ebufs[0].at[1]],
                                  ssem[rslot]).wait()

        edge_dma(0, 0)
        edge_wait(0)
        edge_dma(1, 1)
        gather_dma(0, 0)

        def pipe_body(j4, carry):
            for p in range(4):
                j = j4 * 4 + p
                rs = p % 2
                @pl.when(j > 0)
                def _():
                    scatter_wait(1 - rs)
                edge_dma(jnp.minimum(j + 2, last), (p + 2) % 4)
                edge_wait((p + 1) % 4)
                gather_dma((p + 1) % 4, 1 - rs)
                gather_wait(rs)

                def scale_group(g, gc):
                    vals = vbufs[p][pl.ds(g * L, L)]
                    for ei in range(L):
                        vb = jnp.full((L,), vals[ei], jnp.float32)
                        row = g * L + ei
                        for f in range(fslices):
                            sl = pl.ds(f * L, L)
                            rows[rs][row, sl] = rows[rs][row, sl] * vb
                    return gc

                lax.fori_loop(0, C // L, scale_group, 0)
                scatter_dma(p, rs)
            return carry

        lax.fori_loop(0, n_chunks // 4, pipe_body, 0)

        edge_wait(1)
        gather_wait(0)
        scatter_wait(1)
        plsc.subcore_barrier()

        for k in range(wb_chunks):
            r = row0 + k * C
            pltpu.sync_copy(acc_sh.at[pl.ds(r, C)], rw0)
            pltpu.sync_copy(rw0, out_hbm.at[cid, pl.ds(r, C)])

    partials = sc_scatter(support, edges, vals3)

    return pl.pallas_call(
        _finish_body,
        out_shape=jax.ShapeDtypeStruct((n, d_out), jnp.float32),
    )(partials)
